# baseline (speedup 1.0000x reference)
import jax
import jax.numpy as jnp
from jax import lax
from jax.experimental import pallas as pl
from jax.experimental.pallas import tpu as pltpu
from jax.experimental.pallas import tpu_sc as plsc

_B, _S, _F = 4, 4096, 2048
_L = 16
_W = 256
_G = _W // _L
_RI = 128
_RO = 64
_NCI = _S // _RI
_NCO = _S // _RO
_TASKS_PER_BATCH = _F // _W


def _cumsum_body(x_hbm, out_hbm, in0, in1, ot0, ot1, si0, si1, so0, so1):
    ins, outs, sis, sos = (in0, in1), (ot0, ot1), (si0, si1), (so0, so1)
    core = lax.axis_index("c")
    sub = lax.axis_index("s")
    wid = sub * 2 + core
    b = wid // _TASKS_PER_BATCH
    c0 = (wid % _TASKS_PER_BATCH) * _W

    def src(k):
        return x_hbm.at[b, pl.ds(k * _RI, _RI), pl.ds(c0, _W)]

    def dst(o):
        return out_hbm.at[b, pl.ds(o * _RO, _RO), pl.ds(c0, _W)]

    for s in range(2):
        pltpu.make_async_copy(src(s), ins[s], sis[s]).start()

    def compute(ibuf, row0, obuf, carries):
        def row_body(r, cs):
            res = []
            for g in range(_G):
                c = cs[g] + ibuf[row0 + r, pl.ds(g * _L, _L)]
                obuf[r, pl.ds(g * _L, _L)] = c
                res.append(c)
            return tuple(res)

        return lax.fori_loop(0, _RO, row_body, carries)

    def ring_body(j, carries):
        for s in range(2):
            k = 2 * j + s
            ibuf, si = ins[s], sis[s]
            pltpu.make_async_copy(src(k), ibuf, si).wait()

            for h in range(2):
                o = 2 * k + h
                obuf, so = outs[h], sos[h]

                @pl.when(o >= 2)
                def _():
                    pltpu.make_async_copy(obuf, dst(o), so).wait()

                carries = compute(ibuf, h * _RO, obuf, carries)
                pltpu.make_async_copy(obuf, dst(o), so).start()

            @pl.when(k + 2 < _NCI)
            def _():
                pltpu.make_async_copy(src(k + 2), ibuf, si).start()

        return carries

    zeros = tuple(jnp.zeros((_L,), jnp.float32) for _ in range(_G))
    lax.fori_loop(0, _NCI // 2, ring_body, zeros)

    pltpu.make_async_copy(ot0, dst(_NCO - 2), so0).wait()
    pltpu.make_async_copy(ot1, dst(_NCO - 1), so1).wait()


@jax.jit
def _cumsum_axis1(x):
    mesh = plsc.VectorSubcoreMesh(
        core_axis_name="c", subcore_axis_name="s", num_cores=2, num_subcores=16
    )
    return pl.kernel(
        _cumsum_body,
        out_type=jax.ShapeDtypeStruct((_B, _S, _F), jnp.float32),
        mesh=mesh,
        scratch_types=(
            [pltpu.VMEM((_RI, _W), jnp.float32)] * 2
            + [pltpu.VMEM((_RO, _W), jnp.float32)] * 2
            + [pltpu.SemaphoreType.DMA] * 4
        ),
    )(x)


def kernel(input_tensor, axis):
    del axis
    return _cumsum_axis1(input_tensor)
